# baseline (speedup 1.0000x reference)
import functools

import jax
import jax.numpy as jnp
from jax.experimental import pallas as pl
from jax.experimental.pallas import tpu as pltpu

BN = 128
P = 16


def _proj_kernel(coset_ref, Wq_ref, bq_ref, Wk_ref, bk_ref, Wv_ref, bv_ref,
                 q_ref, kT_ref, v_ref, *, scale):
    x = coset_ref[0]
    q = jax.lax.dot(x, Wq_ref[...], preferred_element_type=jnp.float32)
    q_ref[0] = (q + bq_ref[...]) * scale
    k = jax.lax.dot(x, Wk_ref[...], preferred_element_type=jnp.float32)
    kT_ref[0] = (k + bk_ref[...]).T
    v = jax.lax.dot(x, Wv_ref[...], preferred_element_type=jnp.float32)
    v_ref[0] = (v + bv_ref[...]).astype(jnp.bfloat16)


def _swish(a):
    return a * (0.5 + 0.5 * jnp.tanh(0.5 * a))


def _main_kernel(pg_ref, q_ref, kT_ref, v_ref, mb_ref,
                 W1_ref, b1_ref, W2_ref, b2_ref, W3_ref, b3_ref,
                 Wo_ref, bo_ref, out_ref, *, n, nh, dh, dv):
    f32 = jnp.float32
    bf16 = jnp.bfloat16
    g = n // P

    x = pg_ref[0].reshape(BN * g, P * 6)
    h1 = jax.lax.dot(x, W1_ref[...], preferred_element_type=f32)
    h1 = _swish(h1 + b1_ref[...]).astype(bf16)
    h2 = jax.lax.dot(h1, W2_ref[...], preferred_element_type=f32)
    h2 = _swish(h2 + b2_ref[...]).astype(bf16)
    locp = jax.lax.dot(h2, W3_ref[...], preferred_element_type=f32)
    locp = (locp + b3_ref[...]).reshape(BN, g, nh * P)

    qb = q_ref[0]
    mb = mb_ref[0]
    outs = []
    for h in range(nh):
        loc = locp[:, :, h * P:(h + 1) * P].reshape(BN, n)
        dots = jax.lax.dot(qb[:, h * dh:(h + 1) * dh],
                           kT_ref[0, h * dh:(h + 1) * dh, :],
                           preferred_element_type=f32)
        e = jnp.exp(loc + dots + mb)
        s = jnp.sum(e, axis=-1, keepdims=True)
        ov = jax.lax.dot(e.astype(bf16), v_ref[0, :, h * dv:(h + 1) * dv],
                         preferred_element_type=f32)
        outs.append(ov / s)
    o = jnp.concatenate(outs, axis=-1)
    out_ref[0] = jax.lax.dot(o, Wo_ref[...],
                             preferred_element_type=f32) + bo_ref[...]


def kernel(pairwise_g, coset_functions, mask, loc_W1, loc_b1, loc_W2, loc_b2,
           loc_W3, loc_b3, Wq, bq, Wk, bk, W_in, b_in, W_out, b_out):
    bs, n, d = coset_functions.shape
    hid = loc_b1.shape[0]
    nh = loc_b3.shape[0]
    dh = d // nh
    c_out = b_in.shape[0]
    dv = c_out // nh
    f32 = jnp.float32
    bf16 = jnp.bfloat16
    gd = pairwise_g.shape[-1]
    g = n // P

    pg_p = pairwise_g.reshape(bs, n, g, P * gd).astype(bf16)
    mask_bias = jnp.where(mask, 0.0, -1e38).astype(f32).reshape(bs, 1, n)
    eye = jnp.eye(P, dtype=f32)
    W1bd = jnp.kron(eye, loc_W1).astype(bf16)
    W2bd = jnp.kron(eye, loc_W2).astype(bf16)
    b1t = jnp.tile(loc_b1, P).reshape(1, P * hid)
    b2t = jnp.tile(loc_b2, P).reshape(1, P * hid)
    cp = jnp.arange(nh * P)
    perm = (cp % P) * nh + cp // P
    W3bd = jnp.kron(eye, loc_W3)[:, perm].astype(bf16)
    b3t = jnp.repeat(loc_b3, P).reshape(1, nh * P)

    proj = pl.pallas_call(
        functools.partial(_proj_kernel, scale=1.0 / (dh ** 0.5)),
        grid=(bs,),
        in_specs=[
            pl.BlockSpec((1, n, d), lambda b: (b, 0, 0)),
            pl.BlockSpec((d, d), lambda b: (0, 0)),
            pl.BlockSpec((1, d), lambda b: (0, 0)),
            pl.BlockSpec((d, d), lambda b: (0, 0)),
            pl.BlockSpec((1, d), lambda b: (0, 0)),
            pl.BlockSpec((d, c_out), lambda b: (0, 0)),
            pl.BlockSpec((1, c_out), lambda b: (0, 0)),
        ],
        out_specs=[
            pl.BlockSpec((1, n, d), lambda b: (b, 0, 0)),
            pl.BlockSpec((1, d, n), lambda b: (b, 0, 0)),
            pl.BlockSpec((1, n, c_out), lambda b: (b, 0, 0)),
        ],
        out_shape=[
            jax.ShapeDtypeStruct((bs, n, d), f32),
            jax.ShapeDtypeStruct((bs, d, n), f32),
            jax.ShapeDtypeStruct((bs, n, c_out), bf16),
        ],
    )
    q, kT, v = proj(coset_functions, Wq, bq.reshape(1, d), Wk,
                    bk.reshape(1, d), W_in, b_in.reshape(1, c_out))

    out = pl.pallas_call(
        functools.partial(_main_kernel, n=n, nh=nh, dh=dh, dv=dv),
        grid=(bs, n // BN),
        in_specs=[
            pl.BlockSpec((1, BN, g, P * gd), lambda b, i: (b, i, 0, 0)),
            pl.BlockSpec((1, BN, d), lambda b, i: (b, i, 0)),
            pl.BlockSpec((1, d, n), lambda b, i: (b, 0, 0)),
            pl.BlockSpec((1, n, c_out), lambda b, i: (b, 0, 0)),
            pl.BlockSpec((1, 1, n), lambda b, i: (b, 0, 0)),
            pl.BlockSpec((P * gd, P * hid), lambda b, i: (0, 0)),
            pl.BlockSpec((1, P * hid), lambda b, i: (0, 0)),
            pl.BlockSpec((P * hid, P * hid), lambda b, i: (0, 0)),
            pl.BlockSpec((1, P * hid), lambda b, i: (0, 0)),
            pl.BlockSpec((P * hid, nh * P), lambda b, i: (0, 0)),
            pl.BlockSpec((1, nh * P), lambda b, i: (0, 0)),
            pl.BlockSpec((c_out, c_out), lambda b, i: (0, 0)),
            pl.BlockSpec((1, c_out), lambda b, i: (0, 0)),
        ],
        out_specs=pl.BlockSpec((1, BN, c_out), lambda b, i: (b, i, 0)),
        out_shape=jax.ShapeDtypeStruct((bs, n, c_out), f32),
        compiler_params=pltpu.CompilerParams(
            dimension_semantics=("parallel", "parallel")),
    )(pg_p, q, kT, v, mask_bias,
      W1bd, b1t, W2bd, b2t, W3bd, b3t,
      W_out, b_out.reshape(1, c_out))

    return (pairwise_g, out, mask)


# speedup vs baseline: 1.2309x; 1.2309x over previous
import functools

import jax
import jax.numpy as jnp
from jax.experimental import pallas as pl
from jax.experimental.pallas import tpu as pltpu

BN = 128
P = 16


def _proj_kernel(coset_ref, Wq_ref, bq_ref, Wk_ref, bk_ref, Wv_ref, bv_ref,
                 q_ref, kT_ref, v_ref, *, scale):
    x = coset_ref[0]
    q = jax.lax.dot(x, Wq_ref[...], preferred_element_type=jnp.float32)
    q_ref[0] = ((q + bq_ref[...]) * scale).astype(jnp.bfloat16)
    k = jax.lax.dot(x, Wk_ref[...], preferred_element_type=jnp.float32)
    kT_ref[0] = (k + bk_ref[...]).T.astype(jnp.bfloat16)
    v = jax.lax.dot(x, Wv_ref[...], preferred_element_type=jnp.float32)
    v_ref[0] = (v + bv_ref[...]).astype(jnp.bfloat16)


def _swish(a):
    return a * (0.5 + 0.5 * jnp.tanh(0.5 * a))


def _main_kernel(pg_ref, q_ref, kT_ref, v_ref, mb_ref,
                 W1_ref, b1_ref, W2_ref, b2_ref, W3_ref, b3_ref,
                 Wo_ref, bo_ref, out_ref, *, n, nh, dh, dv):
    f32 = jnp.float32
    bf16 = jnp.bfloat16
    g = n // P

    x = pg_ref[0].reshape(BN * g, P * 6).astype(bf16)
    h1 = jax.lax.dot(x, W1_ref[...], preferred_element_type=f32)
    h1 = _swish(h1 + b1_ref[...]).astype(bf16)
    h2 = jax.lax.dot(h1, W2_ref[...], preferred_element_type=f32)
    h2 = _swish(h2 + b2_ref[...]).astype(bf16)
    locp = jax.lax.dot(h2, W3_ref[...], preferred_element_type=f32)
    locp = (locp + b3_ref[...]).astype(bf16).reshape(BN, g, nh * P)

    qb = q_ref[0]
    mb = mb_ref[0]
    outs = []
    for h in range(nh):
        loc = locp[:, :, h * P:(h + 1) * P].reshape(BN, n)
        dots = jax.lax.dot(qb[:, h * dh:(h + 1) * dh],
                           kT_ref[0, h * dh:(h + 1) * dh, :],
                           preferred_element_type=f32)
        e = jnp.exp(loc.astype(f32) + dots + mb)
        s = jnp.sum(e, axis=-1, keepdims=True)
        ov = jax.lax.dot(e.astype(bf16), v_ref[0, :, h * dv:(h + 1) * dv],
                         preferred_element_type=f32)
        outs.append(ov / s)
    o = jnp.concatenate(outs, axis=-1)
    out_ref[0] = jax.lax.dot(o, Wo_ref[...],
                             preferred_element_type=f32) + bo_ref[...]


def kernel(pairwise_g, coset_functions, mask, loc_W1, loc_b1, loc_W2, loc_b2,
           loc_W3, loc_b3, Wq, bq, Wk, bk, W_in, b_in, W_out, b_out):
    bs, n, d = coset_functions.shape
    hid = loc_b1.shape[0]
    nh = loc_b3.shape[0]
    dh = d // nh
    c_out = b_in.shape[0]
    dv = c_out // nh
    f32 = jnp.float32
    bf16 = jnp.bfloat16
    gd = pairwise_g.shape[-1]
    g = n // P

    pg_p = pairwise_g.reshape(bs, n, g, P * gd)
    mask_bias = jnp.where(mask, 0.0, -1e38).astype(f32).reshape(bs, 1, n)
    eye = jnp.eye(P, dtype=f32)
    W1bd = jnp.kron(eye, loc_W1).astype(bf16)
    W2bd = jnp.kron(eye, loc_W2).astype(bf16)
    b1t = jnp.tile(loc_b1, P).reshape(1, P * hid)
    b2t = jnp.tile(loc_b2, P).reshape(1, P * hid)
    cp = jnp.arange(nh * P)
    perm = (cp % P) * nh + cp // P
    W3bd = jnp.kron(eye, loc_W3)[:, perm].astype(bf16)
    b3t = jnp.repeat(loc_b3, P).reshape(1, nh * P)

    proj = pl.pallas_call(
        functools.partial(_proj_kernel, scale=1.0 / (dh ** 0.5)),
        grid=(bs,),
        in_specs=[
            pl.BlockSpec((1, n, d), lambda b: (b, 0, 0)),
            pl.BlockSpec((d, d), lambda b: (0, 0)),
            pl.BlockSpec((1, d), lambda b: (0, 0)),
            pl.BlockSpec((d, d), lambda b: (0, 0)),
            pl.BlockSpec((1, d), lambda b: (0, 0)),
            pl.BlockSpec((d, c_out), lambda b: (0, 0)),
            pl.BlockSpec((1, c_out), lambda b: (0, 0)),
        ],
        out_specs=[
            pl.BlockSpec((1, n, d), lambda b: (b, 0, 0)),
            pl.BlockSpec((1, d, n), lambda b: (b, 0, 0)),
            pl.BlockSpec((1, n, c_out), lambda b: (b, 0, 0)),
        ],
        out_shape=[
            jax.ShapeDtypeStruct((bs, n, d), bf16),
            jax.ShapeDtypeStruct((bs, d, n), bf16),
            jax.ShapeDtypeStruct((bs, n, c_out), bf16),
        ],
    )
    q, kT, v = proj(coset_functions, Wq, bq.reshape(1, d), Wk,
                    bk.reshape(1, d), W_in, b_in.reshape(1, c_out))

    out = pl.pallas_call(
        functools.partial(_main_kernel, n=n, nh=nh, dh=dh, dv=dv),
        grid=(bs, n // BN),
        in_specs=[
            pl.BlockSpec((1, BN, g, P * gd), lambda b, i: (b, i, 0, 0)),
            pl.BlockSpec((1, BN, d), lambda b, i: (b, i, 0)),
            pl.BlockSpec((1, d, n), lambda b, i: (b, 0, 0)),
            pl.BlockSpec((1, n, c_out), lambda b, i: (b, 0, 0)),
            pl.BlockSpec((1, 1, n), lambda b, i: (b, 0, 0)),
            pl.BlockSpec((P * gd, P * hid), lambda b, i: (0, 0)),
            pl.BlockSpec((1, P * hid), lambda b, i: (0, 0)),
            pl.BlockSpec((P * hid, P * hid), lambda b, i: (0, 0)),
            pl.BlockSpec((1, P * hid), lambda b, i: (0, 0)),
            pl.BlockSpec((P * hid, nh * P), lambda b, i: (0, 0)),
            pl.BlockSpec((1, nh * P), lambda b, i: (0, 0)),
            pl.BlockSpec((c_out, c_out), lambda b, i: (0, 0)),
            pl.BlockSpec((1, c_out), lambda b, i: (0, 0)),
        ],
        out_specs=pl.BlockSpec((1, BN, c_out), lambda b, i: (b, i, 0)),
        out_shape=jax.ShapeDtypeStruct((bs, n, c_out), f32),
        compiler_params=pltpu.CompilerParams(
            dimension_semantics=("parallel", "parallel")),
    )(pg_p, q, kT, v, mask_bias,
      W1bd, b1t, W2bd, b2t, W3bd, b3t,
      W_out, b_out.reshape(1, c_out))

    return (pairwise_g, out, mask)


# speedup vs baseline: 1.4176x; 1.1517x over previous
import functools

import jax
import jax.numpy as jnp
from jax.experimental import pallas as pl
from jax.experimental.pallas import tpu as pltpu

BN = 128
P = 16


def _proj_kernel(coset_ref, Wq_ref, bq_ref, Wk_ref, bk_ref, Wv_ref, bv_ref,
                 q_ref, kT_ref, v_ref, *, scale):
    x = coset_ref[0]
    q = jax.lax.dot(x, Wq_ref[...], preferred_element_type=jnp.float32)
    q_ref[0] = ((q + bq_ref[...]) * scale).astype(jnp.bfloat16)
    k = jax.lax.dot(x, Wk_ref[...], preferred_element_type=jnp.float32)
    kT_ref[0] = (k + bk_ref[...]).T.astype(jnp.bfloat16)
    v = jax.lax.dot(x, Wv_ref[...], preferred_element_type=jnp.float32)
    v_ref[0] = (v + bv_ref[...]).astype(jnp.bfloat16)


def _swish(a):
    return a * (0.5 + 0.5 * jnp.tanh(0.5 * a))


def _main_kernel(pg_ref, q_ref, kT_ref, v_ref, mb_ref,
                 W1_ref, b1_ref, W2_ref, b2_ref, W3_ref, b3_ref,
                 Wo_ref, bo_ref, out_ref, *, n, nh, dh, dv):
    f32 = jnp.float32
    bf16 = jnp.bfloat16
    g = n // P

    x = pg_ref[0].reshape(BN * g, P * 6).astype(bf16)
    h1 = jax.lax.dot(x, W1_ref[...], preferred_element_type=f32)
    h1 = _swish(h1 + b1_ref[...]).astype(bf16)
    h2 = jax.lax.dot(h1, W2_ref[...], preferred_element_type=f32)
    h2 = _swish(h2 + b2_ref[...]).astype(bf16)
    locp = jax.lax.dot(h2, W3_ref[...], preferred_element_type=f32)
    locp = (locp + b3_ref[...]).astype(bf16).reshape(BN, g, nh * P)

    qb = q_ref[0]
    mb = mb_ref[0]
    locs = [locp[:, :, h * P:(h + 1) * P].reshape(BN, n) for h in range(nh)]
    dots = [jax.lax.dot(qb[:, h * dh:(h + 1) * dh],
                        kT_ref[0, h * dh:(h + 1) * dh, :],
                        preferred_element_type=f32) for h in range(nh)]
    es = [jnp.exp(locs[h].astype(f32) + dots[h] + mb).astype(bf16)
          for h in range(nh)]
    outs = []
    for h in range(nh):
        e = es[h]
        s = jnp.sum(e.astype(f32), axis=-1, keepdims=True)
        ov = jax.lax.dot(e, v_ref[0, :, h * dv:(h + 1) * dv],
                         preferred_element_type=f32)
        outs.append(ov / s)
    o = jnp.concatenate(outs, axis=-1)
    out_ref[0] = jax.lax.dot(o, Wo_ref[...],
                             preferred_element_type=f32) + bo_ref[...]


def kernel(pairwise_g, coset_functions, mask, loc_W1, loc_b1, loc_W2, loc_b2,
           loc_W3, loc_b3, Wq, bq, Wk, bk, W_in, b_in, W_out, b_out):
    bs, n, d = coset_functions.shape
    hid = loc_b1.shape[0]
    nh = loc_b3.shape[0]
    dh = d // nh
    c_out = b_in.shape[0]
    dv = c_out // nh
    f32 = jnp.float32
    bf16 = jnp.bfloat16
    gd = pairwise_g.shape[-1]
    g = n // P

    pg_p = pairwise_g.reshape(bs, n, g, P * gd)
    mask_bias = jnp.where(mask, 0.0, -1e38).astype(f32).reshape(bs, 1, n)
    eye = jnp.eye(P, dtype=f32)
    W1bd = jnp.kron(eye, loc_W1).astype(bf16)
    W2bd = jnp.kron(eye, loc_W2).astype(bf16)
    b1t = jnp.tile(loc_b1, P).reshape(1, P * hid)
    b2t = jnp.tile(loc_b2, P).reshape(1, P * hid)
    cp = jnp.arange(nh * P)
    perm = (cp % P) * nh + cp // P
    W3bd = jnp.kron(eye, loc_W3)[:, perm].astype(bf16)
    b3t = jnp.repeat(loc_b3, P).reshape(1, nh * P)

    proj = pl.pallas_call(
        functools.partial(_proj_kernel, scale=1.0 / (dh ** 0.5)),
        grid=(bs,),
        in_specs=[
            pl.BlockSpec((1, n, d), lambda b: (b, 0, 0)),
            pl.BlockSpec((d, d), lambda b: (0, 0)),
            pl.BlockSpec((1, d), lambda b: (0, 0)),
            pl.BlockSpec((d, d), lambda b: (0, 0)),
            pl.BlockSpec((1, d), lambda b: (0, 0)),
            pl.BlockSpec((d, c_out), lambda b: (0, 0)),
            pl.BlockSpec((1, c_out), lambda b: (0, 0)),
        ],
        out_specs=[
            pl.BlockSpec((1, n, d), lambda b: (b, 0, 0)),
            pl.BlockSpec((1, d, n), lambda b: (b, 0, 0)),
            pl.BlockSpec((1, n, c_out), lambda b: (b, 0, 0)),
        ],
        out_shape=[
            jax.ShapeDtypeStruct((bs, n, d), bf16),
            jax.ShapeDtypeStruct((bs, d, n), bf16),
            jax.ShapeDtypeStruct((bs, n, c_out), bf16),
        ],
    )
    q, kT, v = proj(coset_functions, Wq, bq.reshape(1, d), Wk,
                    bk.reshape(1, d), W_in, b_in.reshape(1, c_out))

    out = pl.pallas_call(
        functools.partial(_main_kernel, n=n, nh=nh, dh=dh, dv=dv),
        grid=(bs, n // BN),
        in_specs=[
            pl.BlockSpec((1, BN, g, P * gd), lambda b, i: (b, i, 0, 0)),
            pl.BlockSpec((1, BN, d), lambda b, i: (b, i, 0)),
            pl.BlockSpec((1, d, n), lambda b, i: (b, 0, 0)),
            pl.BlockSpec((1, n, c_out), lambda b, i: (b, 0, 0)),
            pl.BlockSpec((1, 1, n), lambda b, i: (b, 0, 0)),
            pl.BlockSpec((P * gd, P * hid), lambda b, i: (0, 0)),
            pl.BlockSpec((1, P * hid), lambda b, i: (0, 0)),
            pl.BlockSpec((P * hid, P * hid), lambda b, i: (0, 0)),
            pl.BlockSpec((1, P * hid), lambda b, i: (0, 0)),
            pl.BlockSpec((P * hid, nh * P), lambda b, i: (0, 0)),
            pl.BlockSpec((1, nh * P), lambda b, i: (0, 0)),
            pl.BlockSpec((c_out, c_out), lambda b, i: (0, 0)),
            pl.BlockSpec((1, c_out), lambda b, i: (0, 0)),
        ],
        out_specs=pl.BlockSpec((1, BN, c_out), lambda b, i: (b, i, 0)),
        out_shape=jax.ShapeDtypeStruct((bs, n, c_out), f32),
        compiler_params=pltpu.CompilerParams(
            dimension_semantics=("parallel", "parallel")),
    )(pg_p, q, kT, v, mask_bias,
      W1bd, b1t, W2bd, b2t, W3bd, b3t,
      W_out, b_out.reshape(1, c_out))

    return (pairwise_g, out, mask)


# speedup vs baseline: 1.4610x; 1.0306x over previous
import functools

import jax
import jax.numpy as jnp
from jax.experimental import pallas as pl
from jax.experimental.pallas import tpu as pltpu

BN = 256
P = 16


def _proj_kernel(coset_ref, Wq_ref, bq_ref, Wk_ref, bk_ref, Wv_ref, bv_ref,
                 q_ref, kT_ref, v_ref, *, scale):
    x = coset_ref[0]
    q = jax.lax.dot(x, Wq_ref[...], preferred_element_type=jnp.float32)
    q_ref[0] = ((q + bq_ref[...]) * scale).astype(jnp.bfloat16)
    k = jax.lax.dot(x, Wk_ref[...], preferred_element_type=jnp.float32)
    kT_ref[0] = (k + bk_ref[...]).T.astype(jnp.bfloat16)
    v = jax.lax.dot(x, Wv_ref[...], preferred_element_type=jnp.float32)
    v_ref[0] = (v + bv_ref[...]).astype(jnp.bfloat16)


def _swish(a):
    return a * (0.5 + 0.5 * jnp.tanh(0.5 * a))


def _main_kernel(pg_ref, q_ref, kT_ref, v_ref, mb_ref,
                 W1_ref, b1_ref, W2_ref, b2_ref, W3_ref, b3_ref,
                 Wo_ref, bo_ref, out_ref, *, n, nh, dh, dv):
    f32 = jnp.float32
    bf16 = jnp.bfloat16
    g = n // P

    x = pg_ref[0].reshape(BN * g, P * 6).astype(bf16)
    h1 = jax.lax.dot(x, W1_ref[...], preferred_element_type=f32)
    h1 = _swish(h1 + b1_ref[...]).astype(bf16)
    h2 = jax.lax.dot(h1, W2_ref[...], preferred_element_type=f32)
    h2 = _swish(h2 + b2_ref[...]).astype(bf16)
    locp = jax.lax.dot(h2, W3_ref[...], preferred_element_type=f32)
    locp = (locp + b3_ref[...]).astype(bf16).reshape(BN, g, nh * P)

    qb = q_ref[0]
    mb = mb_ref[0]
    locs = [locp[:, :, h * P:(h + 1) * P].reshape(BN, n) for h in range(nh)]
    dots = [jax.lax.dot(qb[:, h * dh:(h + 1) * dh],
                        kT_ref[0, h * dh:(h + 1) * dh, :],
                        preferred_element_type=f32) for h in range(nh)]
    es = [jnp.exp(locs[h].astype(f32) + dots[h] + mb).astype(bf16)
          for h in range(nh)]
    outs = []
    for h in range(nh):
        e = es[h]
        s = jnp.sum(e.astype(f32), axis=-1, keepdims=True)
        ov = jax.lax.dot(e, v_ref[0, :, h * dv:(h + 1) * dv],
                         preferred_element_type=f32)
        outs.append(ov / s)
    o = jnp.concatenate(outs, axis=-1)
    out_ref[0] = jax.lax.dot(o, Wo_ref[...],
                             preferred_element_type=f32) + bo_ref[...]


def kernel(pairwise_g, coset_functions, mask, loc_W1, loc_b1, loc_W2, loc_b2,
           loc_W3, loc_b3, Wq, bq, Wk, bk, W_in, b_in, W_out, b_out):
    bs, n, d = coset_functions.shape
    hid = loc_b1.shape[0]
    nh = loc_b3.shape[0]
    dh = d // nh
    c_out = b_in.shape[0]
    dv = c_out // nh
    f32 = jnp.float32
    bf16 = jnp.bfloat16
    gd = pairwise_g.shape[-1]
    g = n // P

    pg_p = pairwise_g.reshape(bs, n, g, P * gd)
    mask_bias = jnp.where(mask, 0.0, -1e38).astype(f32).reshape(bs, 1, n)
    eye = jnp.eye(P, dtype=f32)
    W1bd = jnp.kron(eye, loc_W1).astype(bf16)
    W2bd = jnp.kron(eye, loc_W2).astype(bf16)
    b1t = jnp.tile(loc_b1, P).reshape(1, P * hid)
    b2t = jnp.tile(loc_b2, P).reshape(1, P * hid)
    cp = jnp.arange(nh * P)
    perm = (cp % P) * nh + cp // P
    W3bd = jnp.kron(eye, loc_W3)[:, perm].astype(bf16)
    b3t = jnp.repeat(loc_b3, P).reshape(1, nh * P)

    proj = pl.pallas_call(
        functools.partial(_proj_kernel, scale=1.0 / (dh ** 0.5)),
        grid=(bs,),
        in_specs=[
            pl.BlockSpec((1, n, d), lambda b: (b, 0, 0)),
            pl.BlockSpec((d, d), lambda b: (0, 0)),
            pl.BlockSpec((1, d), lambda b: (0, 0)),
            pl.BlockSpec((d, d), lambda b: (0, 0)),
            pl.BlockSpec((1, d), lambda b: (0, 0)),
            pl.BlockSpec((d, c_out), lambda b: (0, 0)),
            pl.BlockSpec((1, c_out), lambda b: (0, 0)),
        ],
        out_specs=[
            pl.BlockSpec((1, n, d), lambda b: (b, 0, 0)),
            pl.BlockSpec((1, d, n), lambda b: (b, 0, 0)),
            pl.BlockSpec((1, n, c_out), lambda b: (b, 0, 0)),
        ],
        out_shape=[
            jax.ShapeDtypeStruct((bs, n, d), bf16),
            jax.ShapeDtypeStruct((bs, d, n), bf16),
            jax.ShapeDtypeStruct((bs, n, c_out), bf16),
        ],
    )
    q, kT, v = proj(coset_functions, Wq, bq.reshape(1, d), Wk,
                    bk.reshape(1, d), W_in, b_in.reshape(1, c_out))

    out = pl.pallas_call(
        functools.partial(_main_kernel, n=n, nh=nh, dh=dh, dv=dv),
        grid=(bs, n // BN),
        in_specs=[
            pl.BlockSpec((1, BN, g, P * gd), lambda b, i: (b, i, 0, 0)),
            pl.BlockSpec((1, BN, d), lambda b, i: (b, i, 0)),
            pl.BlockSpec((1, d, n), lambda b, i: (b, 0, 0)),
            pl.BlockSpec((1, n, c_out), lambda b, i: (b, 0, 0)),
            pl.BlockSpec((1, 1, n), lambda b, i: (b, 0, 0)),
            pl.BlockSpec((P * gd, P * hid), lambda b, i: (0, 0)),
            pl.BlockSpec((1, P * hid), lambda b, i: (0, 0)),
            pl.BlockSpec((P * hid, P * hid), lambda b, i: (0, 0)),
            pl.BlockSpec((1, P * hid), lambda b, i: (0, 0)),
            pl.BlockSpec((P * hid, nh * P), lambda b, i: (0, 0)),
            pl.BlockSpec((1, nh * P), lambda b, i: (0, 0)),
            pl.BlockSpec((c_out, c_out), lambda b, i: (0, 0)),
            pl.BlockSpec((1, c_out), lambda b, i: (0, 0)),
        ],
        out_specs=pl.BlockSpec((1, BN, c_out), lambda b, i: (b, i, 0)),
        out_shape=jax.ShapeDtypeStruct((bs, n, c_out), f32),
        compiler_params=pltpu.CompilerParams(
            dimension_semantics=("parallel", "parallel")),
    )(pg_p, q, kT, v, mask_bias,
      W1bd, b1t, W2bd, b2t, W3bd, b3t,
      W_out, b_out.reshape(1, c_out))

    return (pairwise_g, out, mask)


# speedup vs baseline: 1.5024x; 1.0283x over previous
import functools

import jax
import jax.numpy as jnp
from jax.experimental import pallas as pl
from jax.experimental.pallas import tpu as pltpu

BN = 128
P = 16


def _proj_kernel(coset_ref, Wq_ref, bq_ref, Wk_ref, bk_ref, Wv_ref, bv_ref,
                 q_ref, Kbd_ref, Vbd_ref, *, scale, nh, dh, dv):
    f32 = jnp.float32
    bf16 = jnp.bfloat16
    x = coset_ref[0]
    n = x.shape[0]
    d = nh * dh
    g = n // P
    q = jax.lax.dot(x, Wq_ref[...], preferred_element_type=f32)
    q_ref[0] = ((q + bq_ref[...]) * scale).astype(bf16)
    k = jax.lax.dot(x, Wk_ref[...], preferred_element_type=f32)
    kT = (k + bk_ref[...]).T.astype(bf16)
    v = (jax.lax.dot(x, Wv_ref[...], preferred_element_type=f32)
         + bv_ref[...]).astype(bf16)
    r_i = jax.lax.broadcasted_iota(jnp.int32, (d, d), 0) // dh
    c_i = jax.lax.broadcasted_iota(jnp.int32, (d, d), 1) // dh
    bm = r_i == c_i
    zero = jnp.zeros((d, d), bf16)
    for gg in range(g):
        kTg = kT[:, gg * P:(gg + 1) * P]
        Kbd_ref[0, :, gg * d:(gg + 1) * d] = jnp.where(
            bm, jnp.tile(kTg, (1, nh)), zero)
        vg = v[gg * P:(gg + 1) * P, :]
        Vbd_ref[0, gg * d:(gg + 1) * d, :] = jnp.where(
            bm, jnp.tile(vg, (nh, 1)), zero)


def _swish(a):
    return a * (0.5 + 0.5 * jnp.tanh(0.5 * a))


def _main_kernel(pg_ref, q_ref, Kbd_ref, Vbd_ref, mb_ref,
                 W1_ref, b1_ref, W2_ref, b2_ref, W3_ref, b3_ref,
                 hm_ref, Wo_ref, bo_ref, out_ref, *, n, gd, nh, dh, dv):
    f32 = jnp.float32
    bf16 = jnp.bfloat16
    g = n // P
    w = P * gd

    xf = pg_ref[0].astype(bf16)
    x = jnp.concatenate(
        [xf[:, s * w:(s + 1) * w] for s in range(g)], axis=0)

    h1 = jax.lax.dot(x, W1_ref[...], preferred_element_type=f32)
    h1 = _swish(h1 + b1_ref[...]).astype(bf16)
    h2 = jax.lax.dot(h1, W2_ref[...], preferred_element_type=f32)
    h2 = _swish(h2 + b2_ref[...]).astype(bf16)
    locp = jax.lax.dot(h2, W3_ref[...], preferred_element_type=f32)
    locp = locp + b3_ref[...]

    l3 = locp.reshape(g, BN, nh * P)
    loc_all = jnp.concatenate([l3[gg] for gg in range(g)], axis=1)

    qb = q_ref[0]
    dots_all = jax.lax.dot(qb, Kbd_ref[0], preferred_element_type=f32)
    e_all = jnp.exp(loc_all + dots_all + mb_ref[0])

    d = nh * dh
    r1 = e_all[:, 0:d]
    for gg in range(1, g):
        r1 = r1 + e_all[:, gg * d:(gg + 1) * d]
    srep = jax.lax.dot(r1, hm_ref[...], preferred_element_type=f32)

    acc = jax.lax.dot(e_all.astype(bf16), Vbd_ref[0],
                      preferred_element_type=f32)
    out_ref[0] = jax.lax.dot(acc / srep, Wo_ref[...],
                             preferred_element_type=f32) + bo_ref[...]


def kernel(pairwise_g, coset_functions, mask, loc_W1, loc_b1, loc_W2, loc_b2,
           loc_W3, loc_b3, Wq, bq, Wk, bk, W_in, b_in, W_out, b_out):
    bs, n, d = coset_functions.shape
    hid = loc_b1.shape[0]
    nh = loc_b3.shape[0]
    dh = d // nh
    c_out = b_in.shape[0]
    dv = c_out // nh
    f32 = jnp.float32
    bf16 = jnp.bfloat16
    gd = pairwise_g.shape[-1]
    g = n // P

    pg_f = pairwise_g.reshape(bs, n, n * gd)
    bias = jnp.where(mask, 0.0, -1e38).astype(f32)
    mbp = jnp.tile(bias.reshape(bs, g, P), (1, 1, nh)).reshape(bs, 1, g * d)
    eye = jnp.eye(P, dtype=f32)
    W1bd = jnp.kron(eye, loc_W1).astype(bf16)
    W2bd = jnp.kron(eye, loc_W2).astype(bf16)
    b1t = jnp.tile(loc_b1, P).reshape(1, P * hid)
    b2t = jnp.tile(loc_b2, P).reshape(1, P * hid)
    cp = jnp.arange(nh * P)
    perm = (cp % P) * nh + cp // P
    W3bd = jnp.kron(eye, loc_W3)[:, perm].astype(bf16)
    b3t = jnp.repeat(loc_b3, P).reshape(1, nh * P)
    hmask = jnp.kron(jnp.eye(nh, dtype=f32), jnp.ones((dh, dh), f32))

    proj = pl.pallas_call(
        functools.partial(_proj_kernel, scale=1.0 / (dh ** 0.5),
                          nh=nh, dh=dh, dv=dv),
        grid=(bs,),
        in_specs=[
            pl.BlockSpec((1, n, d), lambda b: (b, 0, 0)),
            pl.BlockSpec((d, d), lambda b: (0, 0)),
            pl.BlockSpec((1, d), lambda b: (0, 0)),
            pl.BlockSpec((d, d), lambda b: (0, 0)),
            pl.BlockSpec((1, d), lambda b: (0, 0)),
            pl.BlockSpec((d, c_out), lambda b: (0, 0)),
            pl.BlockSpec((1, c_out), lambda b: (0, 0)),
        ],
        out_specs=[
            pl.BlockSpec((1, n, d), lambda b: (b, 0, 0)),
            pl.BlockSpec((1, d, g * d), lambda b: (b, 0, 0)),
            pl.BlockSpec((1, g * d, c_out), lambda b: (b, 0, 0)),
        ],
        out_shape=[
            jax.ShapeDtypeStruct((bs, n, d), bf16),
            jax.ShapeDtypeStruct((bs, d, g * d), bf16),
            jax.ShapeDtypeStruct((bs, g * d, c_out), bf16),
        ],
    )
    q, Kbd, Vbd = proj(coset_functions, Wq, bq.reshape(1, d), Wk,
                       bk.reshape(1, d), W_in, b_in.reshape(1, c_out))

    out = pl.pallas_call(
        functools.partial(_main_kernel, n=n, gd=gd, nh=nh, dh=dh, dv=dv),
        grid=(bs, n // BN),
        in_specs=[
            pl.BlockSpec((1, BN, n * gd), lambda b, i: (b, i, 0)),
            pl.BlockSpec((1, BN, d), lambda b, i: (b, i, 0)),
            pl.BlockSpec((1, d, g * d), lambda b, i: (b, 0, 0)),
            pl.BlockSpec((1, g * d, c_out), lambda b, i: (b, 0, 0)),
            pl.BlockSpec((1, 1, g * d), lambda b, i: (b, 0, 0)),
            pl.BlockSpec((P * gd, P * hid), lambda b, i: (0, 0)),
            pl.BlockSpec((1, P * hid), lambda b, i: (0, 0)),
            pl.BlockSpec((P * hid, P * hid), lambda b, i: (0, 0)),
            pl.BlockSpec((1, P * hid), lambda b, i: (0, 0)),
            pl.BlockSpec((P * hid, nh * P), lambda b, i: (0, 0)),
            pl.BlockSpec((1, nh * P), lambda b, i: (0, 0)),
            pl.BlockSpec((d, d), lambda b, i: (0, 0)),
            pl.BlockSpec((c_out, c_out), lambda b, i: (0, 0)),
            pl.BlockSpec((1, c_out), lambda b, i: (0, 0)),
        ],
        out_specs=pl.BlockSpec((1, BN, c_out), lambda b, i: (b, i, 0)),
        out_shape=jax.ShapeDtypeStruct((bs, n, c_out), f32),
        compiler_params=pltpu.CompilerParams(
            dimension_semantics=("parallel", "parallel")),
    )(pg_f, q, Kbd, Vbd, mbp,
      W1bd, b1t, W2bd, b2t, W3bd, b3t,
      hmask, W_out, b_out.reshape(1, c_out))

    return (pairwise_g, out, mask)
